# baseline (speedup 1.0000x reference)
import jax
import jax.numpy as jnp
from jax.experimental import pallas as pl
from jax.experimental.pallas import tpu as pltpu

_SCALE = 8.0 / 8.0

_TN = 512


def _qkv_lora_body(idx_ref, x_ref, w_ref, bias_ref, aq_ref, bq_ref, av_ref,
                   bv_ref, o_ref, ac_ref, bc_ref, we_ref):
    b = pl.program_id(0)
    n = pl.program_id(1)
    D = x_ref.shape[2]
    R = aq_ref.shape[2]

    @pl.when(jnp.logical_and(b == 0, n == 0))
    def _():
        bc_ref[...] = jnp.zeros_like(bc_ref)

    @pl.when(n == 0)
    def _():
        ac_ref[:, :R] = aq_ref[0].astype(jnp.bfloat16)
        ac_ref[:, R:] = av_ref[0].astype(jnp.bfloat16)
        bc_ref[:R, :D] = (_SCALE * bq_ref[0]).astype(jnp.bfloat16)
        bc_ref[R:, 2 * D:] = (_SCALE * bv_ref[0]).astype(jnp.bfloat16)
        upd = jax.lax.dot_general(
            bc_ref[...], ac_ref[...], (((0,), (1,)), ((), ())),
            preferred_element_type=jnp.float32)
        we_ref[...] = w_ref[...] + upd.astype(jnp.bfloat16)

    xb = x_ref[0].astype(jnp.bfloat16)
    acc = jax.lax.dot_general(
        xb, we_ref[...], (((1,), (1,)), ((), ())),
        preferred_element_type=jnp.float32)
    o_ref[0] = acc + bias_ref[...]


def kernel(x, weight, bias, A_q_pool, B_q_pool, A_v_pool, B_v_pool, idx,
           frozen_mask):
    B, N, D = x.shape
    O = weight.shape[0]
    P, _, R = A_q_pool.shape

    idx32 = idx[:, 0].astype(jnp.int32)
    bias2 = bias.reshape(1, O)
    wb = weight.astype(jnp.bfloat16)

    grid = (B, N // _TN)
    grid_spec = pltpu.PrefetchScalarGridSpec(
        num_scalar_prefetch=1,
        grid=grid,
        in_specs=[
            pl.BlockSpec((1, _TN, D), lambda b, n, idx_ref: (b, n, 0)),
            pl.BlockSpec((O, D), lambda b, n, idx_ref: (0, 0)),
            pl.BlockSpec((1, O), lambda b, n, idx_ref: (0, 0)),
            pl.BlockSpec((1, D, R), lambda b, n, idx_ref: (idx_ref[b], 0, 0)),
            pl.BlockSpec((1, R, D), lambda b, n, idx_ref: (idx_ref[b], 0, 0)),
            pl.BlockSpec((1, D, R), lambda b, n, idx_ref: (idx_ref[b], 0, 0)),
            pl.BlockSpec((1, R, D), lambda b, n, idx_ref: (idx_ref[b], 0, 0)),
        ],
        out_specs=pl.BlockSpec((1, _TN, O), lambda b, n, idx_ref: (b, n, 0)),
        scratch_shapes=[
            pltpu.VMEM((D, 2 * R), jnp.bfloat16),
            pltpu.VMEM((2 * R, O), jnp.bfloat16),
            pltpu.VMEM((O, D), jnp.bfloat16),
        ],
    )

    out = pl.pallas_call(
        _qkv_lora_body,
        out_shape=jax.ShapeDtypeStruct((B, N, O), jnp.float32),
        grid_spec=grid_spec,
        compiler_params=pltpu.CompilerParams(
            dimension_semantics=("parallel", "arbitrary"),
            vmem_limit_bytes=56 * 1024 * 1024,
        ),
        name="qkv_lora_fused",
    )(idx32, x, wb, bias2, A_q_pool, B_q_pool, A_v_pool, B_v_pool)
    return out


# speedup vs baseline: 1.0539x; 1.0539x over previous
import jax
import jax.numpy as jnp
from jax.experimental import pallas as pl
from jax.experimental.pallas import tpu as pltpu

_SCALE = 8.0 / 8.0

_TN = 1024


def _qkv_lora_body(idx_ref, x_ref, w_ref, bias_ref, aq_ref, bq_ref, av_ref,
                   bv_ref, o_ref, ac_ref, bc_ref, we_ref):
    b = pl.program_id(0)
    n = pl.program_id(1)
    D = x_ref.shape[2]
    R = aq_ref.shape[2]

    @pl.when(jnp.logical_and(b == 0, n == 0))
    def _():
        bc_ref[...] = jnp.zeros_like(bc_ref)

    @pl.when(n == 0)
    def _():
        ac_ref[:, :R] = aq_ref[0].astype(jnp.bfloat16)
        ac_ref[:, R:] = av_ref[0].astype(jnp.bfloat16)
        bc_ref[:R, :D] = (_SCALE * bq_ref[0]).astype(jnp.bfloat16)
        bc_ref[R:, 2 * D:] = (_SCALE * bv_ref[0]).astype(jnp.bfloat16)
        upd = jax.lax.dot_general(
            bc_ref[...], ac_ref[...], (((0,), (1,)), ((), ())),
            preferred_element_type=jnp.float32)
        we_ref[...] = w_ref[...] + upd.astype(jnp.bfloat16)

    xb = x_ref[0].astype(jnp.bfloat16)
    acc = jax.lax.dot_general(
        xb, we_ref[...], (((1,), (1,)), ((), ())),
        preferred_element_type=jnp.float32)
    o_ref[0] = acc + bias_ref[...]


def kernel(x, weight, bias, A_q_pool, B_q_pool, A_v_pool, B_v_pool, idx,
           frozen_mask):
    B, N, D = x.shape
    O = weight.shape[0]
    P, _, R = A_q_pool.shape

    idx32 = idx[:, 0].astype(jnp.int32)
    bias2 = bias.reshape(1, O)
    wb = weight.astype(jnp.bfloat16)

    grid = (B, N // _TN)
    grid_spec = pltpu.PrefetchScalarGridSpec(
        num_scalar_prefetch=1,
        grid=grid,
        in_specs=[
            pl.BlockSpec((1, _TN, D), lambda b, n, idx_ref: (b, n, 0)),
            pl.BlockSpec((O, D), lambda b, n, idx_ref: (0, 0)),
            pl.BlockSpec((1, O), lambda b, n, idx_ref: (0, 0)),
            pl.BlockSpec((1, D, R), lambda b, n, idx_ref: (idx_ref[b], 0, 0)),
            pl.BlockSpec((1, R, D), lambda b, n, idx_ref: (idx_ref[b], 0, 0)),
            pl.BlockSpec((1, D, R), lambda b, n, idx_ref: (idx_ref[b], 0, 0)),
            pl.BlockSpec((1, R, D), lambda b, n, idx_ref: (idx_ref[b], 0, 0)),
        ],
        out_specs=pl.BlockSpec((1, _TN, O), lambda b, n, idx_ref: (b, n, 0)),
        scratch_shapes=[
            pltpu.VMEM((D, 2 * R), jnp.bfloat16),
            pltpu.VMEM((2 * R, O), jnp.bfloat16),
            pltpu.VMEM((O, D), jnp.bfloat16),
        ],
    )

    out = pl.pallas_call(
        _qkv_lora_body,
        out_shape=jax.ShapeDtypeStruct((B, N, O), jnp.float32),
        grid_spec=grid_spec,
        compiler_params=pltpu.CompilerParams(
            dimension_semantics=("parallel", "arbitrary"),
            vmem_limit_bytes=56 * 1024 * 1024,
        ),
        name="qkv_lora_fused",
    )(idx32, x, wb, bias2, A_q_pool, B_q_pool, A_v_pool, B_v_pool)
    return out
